# baseline (speedup 1.0000x reference)
import functools

import jax
import jax.numpy as jnp
from jax import lax
from jax.experimental import pallas as pl
from jax.experimental.pallas import tpu as pltpu
from jax.experimental.pallas import tpu_sc as plsc


TM = 512
RB = 128
CHC = 128


def _esq_body(e_ref, esq_ref):
    e0 = e_ref[...]
    s = jnp.sum(e0 * e0, axis=1)
    esq_ref[...] = s[None, :]


def _esq_call(emb):
    ne, dim = emb.shape
    return pl.pallas_call(
        _esq_body,
        out_shape=jax.ShapeDtypeStruct((1, ne), jnp.float32),
    )(emb)


def _argmin_body(x_ref, et_ref, esq_ref, idx_ref):
    x2 = x_ref[...] * -2.0
    d = lax.dot_general(
        x2, et_ref[...], (((1,), (1,)), ((), ())),
        preferred_element_type=jnp.float32,
        precision=lax.Precision.DEFAULT,
    )
    x_sq = jnp.sum(x2 * x2, axis=1, keepdims=True) * 0.25
    ne = d.shape[1]

    lane = lax.broadcasted_iota(jnp.int32, (RB, CHC), 1).astype(jnp.float32)
    for r in range(TM // RB):
        xsq_r = x_sq[r * RB:(r + 1) * RB, :]
        m128 = None
        c128 = None
        for c in range(ne // CHC):
            sc = (xsq_r + esq_ref[:, c * CHC:(c + 1) * CHC]) \
                + d[r * RB:(r + 1) * RB, c * CHC:(c + 1) * CHC]
            if m128 is None:
                m128 = sc
                c128 = jnp.zeros((RB, CHC), jnp.float32)
            else:
                better = sc < m128
                m128 = jnp.minimum(sc, m128)
                c128 = jnp.where(better, float(c), c128)

        j128 = c128 * float(CHC) + lane
        m = jnp.min(m128, axis=1, keepdims=True)
        first = jnp.min(jnp.where(m128 == m, j128, float(ne)), axis=1)
        idx_ref[0, 0, r * RB:(r + 1) * RB] = first.astype(jnp.int32)


def _argmin_call(flat_x, emb, esq, tok0, ntok):
    n, dim = flat_x.shape
    ne = emb.shape[0]
    grid = ntok // TM
    blk0 = tok0 // TM
    return pl.pallas_call(
        _argmin_body,
        grid=(grid,),
        in_specs=[
            pl.BlockSpec((TM, dim), lambda i: (blk0 + i, 0)),
            pl.BlockSpec((ne, dim), lambda i: (0, 0)),
            pl.BlockSpec((1, ne), lambda i: (0, 0)),
        ],
        out_specs=pl.BlockSpec((1, 1, TM), lambda i: (i, 0, 0)),
        out_shape=jax.ShapeDtypeStruct((grid, 1, TM), jnp.int32),
        compiler_params=pltpu.CompilerParams(
            dimension_semantics=("arbitrary",),
        ),
    )(flat_x, emb, esq)


NC, NS = 2, 16
NW = NC * NS
CH = 128


def _gather_call(table, idx3):
    nw, nch, ch = idx3.shape
    b = nw * nch * ch
    d = table.shape[1]
    b_per_w = nch * ch
    mesh = plsc.VectorSubcoreMesh(core_axis_name="c", subcore_axis_name="s")

    @functools.partial(
        pl.kernel, mesh=mesh,
        out_type=jax.ShapeDtypeStruct((b, d), jnp.float32),
        scratch_types=[
            pltpu.VMEM((nch, ch), jnp.int32),
            pltpu.VMEM((2, ch, d), jnp.float32),
            pltpu.SemaphoreType.DMA((2,)),
        ],
    )
    def gather_k(table_hbm, idx_hbm, out_hbm, idx_v, rows_v, sems):
        wid = lax.axis_index("s") * NC + lax.axis_index("c")
        pltpu.sync_copy(idx_hbm.at[wid], idx_v)

        def start(c, slot):
            return pltpu.async_copy(
                table_hbm.at[idx_v.at[c]], rows_v.at[slot], sems.at[slot])

        handles = [None] * nch
        handles[0] = start(0, 0)
        for c in range(nch):
            if c + 1 < nch:
                handles[c + 1] = start(c + 1, (c + 1) % 2)
            handles[c].wait()
            pltpu.sync_copy(
                rows_v.at[c % 2],
                out_hbm.at[pl.ds(wid * b_per_w + c * ch, ch)])

    return gather_k(table, idx3)


NSLICES = 8


def kernel(flat_x, embedding):
    n, dim = flat_x.shape
    esq = _esq_call(embedding)
    ns = n // NSLICES
    idx_parts, q_parts = [], []
    for s in range(NSLICES):
        idx3 = _argmin_call(flat_x, embedding, esq, s * ns, ns)
        idx_parts.append(idx3.reshape(ns))
        nch = ns // (NW * CH)
        q_parts.append(_gather_call(embedding, idx3.reshape(NW, nch, CH)))
    indices = jnp.concatenate(idx_parts)
    quantized = jnp.concatenate(q_parts)
    return quantized, indices


# speedup vs baseline: 1.0558x; 1.0558x over previous
import functools

import jax
import jax.numpy as jnp
from jax import lax
from jax.experimental import pallas as pl
from jax.experimental.pallas import tpu as pltpu
from jax.experimental.pallas import tpu_sc as plsc


TM = 512
RB = 128
CHC = 128


def _esq_body(e_ref, esq_ref):
    e0 = e_ref[...]
    s = jnp.sum(e0 * e0, axis=1)
    esq_ref[...] = s[None, :]


def _esq_call(emb):
    ne, dim = emb.shape
    return pl.pallas_call(
        _esq_body,
        out_shape=jax.ShapeDtypeStruct((1, ne), jnp.float32),
    )(emb)


def _argmin_body(x_ref, et_ref, esq_ref, idx_ref):
    x2 = x_ref[...] * -2.0
    d = lax.dot_general(
        x2, et_ref[...], (((1,), (1,)), ((), ())),
        preferred_element_type=jnp.float32,
        precision=lax.Precision.DEFAULT,
    )
    x_sq = jnp.sum(x2 * x2, axis=1, keepdims=True) * 0.25
    ne = d.shape[1]

    lane = lax.broadcasted_iota(jnp.int32, (RB, CHC), 1).astype(jnp.float32)
    for r in range(TM // RB):
        xsq_r = x_sq[r * RB:(r + 1) * RB, :]
        m128 = None
        c128 = None
        for c in range(ne // CHC):
            sc = (xsq_r + esq_ref[:, c * CHC:(c + 1) * CHC]) \
                + d[r * RB:(r + 1) * RB, c * CHC:(c + 1) * CHC]
            if m128 is None:
                m128 = sc
                c128 = jnp.zeros((RB, CHC), jnp.float32)
            else:
                better = sc < m128
                m128 = jnp.minimum(sc, m128)
                c128 = jnp.where(better, float(c), c128)

        j128 = c128 * float(CHC) + lane
        m = jnp.min(m128, axis=1, keepdims=True)
        first = jnp.min(jnp.where(m128 == m, j128, float(ne)), axis=1)
        idx_ref[0, 0, r * RB:(r + 1) * RB] = first.astype(jnp.int32)


def _argmin_call(flat_x, emb, esq, tok0, ntok):
    n, dim = flat_x.shape
    ne = emb.shape[0]
    grid = ntok // TM
    blk0 = tok0 // TM
    return pl.pallas_call(
        _argmin_body,
        grid=(grid,),
        in_specs=[
            pl.BlockSpec((TM, dim), lambda i: (blk0 + i, 0)),
            pl.BlockSpec((ne, dim), lambda i: (0, 0)),
            pl.BlockSpec((1, ne), lambda i: (0, 0)),
        ],
        out_specs=pl.BlockSpec((1, 1, TM), lambda i: (i, 0, 0)),
        out_shape=jax.ShapeDtypeStruct((grid, 1, TM), jnp.int32),
        compiler_params=pltpu.CompilerParams(
            dimension_semantics=("arbitrary",),
        ),
    )(flat_x, emb, esq)


NC, NS = 2, 16
NW = NC * NS
CH = 128


def _gather_call(table, idx3):
    nw, nch, ch = idx3.shape
    b = nw * nch * ch
    d = table.shape[1]
    b_per_w = nch * ch
    mesh = plsc.VectorSubcoreMesh(core_axis_name="c", subcore_axis_name="s")

    @functools.partial(
        pl.kernel, mesh=mesh,
        out_type=jax.ShapeDtypeStruct((b, d), jnp.float32),
        scratch_types=[
            pltpu.VMEM((nch, ch), jnp.int32),
            pltpu.VMEM((2, ch, d), jnp.float32),
            pltpu.SemaphoreType.DMA((2,)),
        ],
    )
    def gather_k(table_hbm, idx_hbm, out_hbm, idx_v, rows_v, sems):
        wid = lax.axis_index("s") * NC + lax.axis_index("c")
        pltpu.sync_copy(idx_hbm.at[wid], idx_v)

        def start(c, slot):
            return pltpu.async_copy(
                table_hbm.at[idx_v.at[c]], rows_v.at[slot], sems.at[slot])

        handles = [None] * nch
        handles[0] = start(0, 0)
        for c in range(nch):
            if c + 1 < nch:
                handles[c + 1] = start(c + 1, (c + 1) % 2)
            handles[c].wait()
            pltpu.sync_copy(
                rows_v.at[c % 2],
                out_hbm.at[pl.ds(wid * b_per_w + c * ch, ch)])

    return gather_k(table, idx3)


NSLICES = 4


def kernel(flat_x, embedding):
    n, dim = flat_x.shape
    esq = _esq_call(embedding)
    ns = n // NSLICES
    idx_parts, q_parts = [], []
    for s in range(NSLICES):
        idx3 = _argmin_call(flat_x, embedding, esq, s * ns, ns)
        idx_parts.append(idx3.reshape(ns))
        nch = ns // (NW * CH)
        q_parts.append(_gather_call(embedding, idx3.reshape(NW, nch, CH)))
    indices = jnp.concatenate(idx_parts)
    quantized = jnp.concatenate(q_parts)
    return quantized, indices
